# baseline (speedup 1.0000x reference)
import functools

import jax
import jax.numpy as jnp
from jax.experimental import pallas as pl
from jax.experimental.pallas import tpu as pltpu


def _dense_body(x_ref, w_ref, o_ref):
    o_ref[...] = jnp.dot(
        x_ref[...].astype(jnp.bfloat16),
        w_ref[...].astype(jnp.bfloat16),
        preferred_element_type=jnp.float32,
    )


def _dense(x, w):
    n = x.shape[0]
    h = w.shape[1]
    return pl.pallas_call(
        _dense_body,
        out_shape=jax.ShapeDtypeStruct((n, h), jnp.float32),
    )(x, w)


def _masked_adj(adj_ref, k, nk, valid_last):
    blk = adj_ref[...]
    if valid_last != blk.shape[1]:
        col = jax.lax.broadcasted_iota(jnp.int32, blk.shape, 1)
        blk = jnp.where((k < nk - 1) | (col < valid_last), blk, 0.0)
    return blk.astype(jnp.bfloat16)


def _phase_a_body(adj_ref, s1_ref, b1_ref, w2_ref, x1_ref, x2p_ref, s2_ref,
                  *, bi, bk, nk, valid_last):
    i = pl.program_id(0)
    k = pl.program_id(1)
    adj_bf = _masked_adj(adj_ref, k, nk, valid_last)

    @pl.when((i == 0) & (k == 0))
    def _():
        s2_ref[...] = jnp.zeros_like(s2_ref)

    part1 = jnp.dot(adj_bf, s1_ref[pl.ds(k * bk, bk), :].astype(jnp.bfloat16),
                    preferred_element_type=jnp.float32)

    @pl.when(k == 0)
    def _():
        x1_ref[...] = part1
        x2p_ref[...] = jnp.zeros_like(x2p_ref)

    @pl.when(k > 0)
    def _():
        x1_ref[...] = x1_ref[...] + part1

    @pl.when((k + 1) * bk <= i * bi)
    def _():
        x2p_ref[...] = x2p_ref[...] + jnp.dot(
            adj_bf, s2_ref[pl.ds(k * bk, bk), :].astype(jnp.bfloat16),
            preferred_element_type=jnp.float32)

    @pl.when(k == nk - 1)
    def _():
        x1 = jnp.maximum(x1_ref[...] + b1_ref[...], 0.0)
        x1_ref[...] = x1
        s2_ref[pl.ds(i * bi, bi), :] = jnp.dot(
            x1.astype(jnp.bfloat16), w2_ref[...].astype(jnp.bfloat16),
            preferred_element_type=jnp.float32)


def _phase_b_body(adj_ref, s2_ref, x2p_ref, b2_ref, x2_ref,
                  *, bi, bk, nk, valid_last):
    i = pl.program_id(0)
    k = pl.program_id(1)
    kb = (i * bi) // bk

    @pl.when(k == kb)
    def _():
        x2_ref[...] = x2p_ref[...] + b2_ref[...]

    @pl.when(k >= kb)
    def _():
        adj_bf = _masked_adj(adj_ref, k, nk, valid_last)
        x2_ref[...] = x2_ref[...] + jnp.dot(
            adj_bf, s2_ref[pl.ds(k * bk, bk), :].astype(jnp.bfloat16),
            preferred_element_type=jnp.float32)


def gcn2(x, adj, W1, b1, W2, b2, bi=400, bk=1024):
    n = adj.shape[0]
    h1 = W1.shape[1]
    h2 = W2.shape[1]
    ni = n // bi
    nk = -(-n // bk)
    npad = nk * bk
    valid_last = n - (nk - 1) * bk

    s1 = _dense(x, W1)
    s1p = jnp.pad(s1, ((0, npad - n), (0, 0)))

    x1, x2p, s2p = pl.pallas_call(
        functools.partial(_phase_a_body, bi=bi, bk=bk, nk=nk,
                          valid_last=valid_last),
        grid=(ni, nk),
        in_specs=[
            pl.BlockSpec((bi, bk), lambda i, k: (i, k)),
            pl.BlockSpec((npad, h1), lambda i, k: (0, 0)),
            pl.BlockSpec((1, h1), lambda i, k: (0, 0)),
            pl.BlockSpec((h1, h2), lambda i, k: (0, 0)),
        ],
        out_specs=[
            pl.BlockSpec((bi, h1), lambda i, k: (i, 0)),
            pl.BlockSpec((bi, h2), lambda i, k: (i, 0)),
            pl.BlockSpec((npad, h2), lambda i, k: (0, 0)),
        ],
        out_shape=[
            jax.ShapeDtypeStruct((n, h1), jnp.float32),
            jax.ShapeDtypeStruct((n, h2), jnp.float32),
            jax.ShapeDtypeStruct((npad, h2), jnp.float32),
        ],
        compiler_params=pltpu.CompilerParams(
            dimension_semantics=("arbitrary", "arbitrary")
        ),
    )(adj, s1p, b1.reshape(1, -1), W2)

    x2 = pl.pallas_call(
        functools.partial(_phase_b_body, bi=bi, bk=bk, nk=nk,
                          valid_last=valid_last),
        grid=(ni, nk),
        in_specs=[
            pl.BlockSpec(
                (bi, bk),
                lambda i, k: (i, jnp.maximum(k, (i * bi) // bk))),
            pl.BlockSpec((npad, h2), lambda i, k: (0, 0)),
            pl.BlockSpec((bi, h2), lambda i, k: (i, 0)),
            pl.BlockSpec((1, h2), lambda i, k: (0, 0)),
        ],
        out_specs=pl.BlockSpec((bi, h2), lambda i, k: (i, 0)),
        out_shape=jax.ShapeDtypeStruct((n, h2), jnp.float32),
        compiler_params=pltpu.CompilerParams(
            dimension_semantics=("arbitrary", "arbitrary")
        ),
    )(adj, s2p, x2p, b2.reshape(1, -1))

    return (x1, x2)


def kernel(x, adj, W1, b1, W2, b2):
    return gcn2(x, adj, W1, b1, W2, b2, bi=400, bk=1024)


# speedup vs baseline: 1.7185x; 1.7185x over previous
import functools

import jax
import jax.numpy as jnp
from jax.experimental import pallas as pl
from jax.experimental.pallas import tpu as pltpu


def _dense_body(x_ref, w_ref, o_ref):
    o_ref[...] = jnp.dot(
        x_ref[...], w_ref[...],
        preferred_element_type=jnp.float32,
        precision=jax.lax.Precision.DEFAULT,
    )


def _dense(x, w):
    n = x.shape[0]
    h = w.shape[1]
    return pl.pallas_call(
        _dense_body,
        out_shape=jax.ShapeDtypeStruct((n, h), jnp.float32),
    )(x, w)


def _agg_body(adj_ref, s_ref, b_ref, o_ref, *, relu):
    acc = jnp.dot(
        adj_ref[...], s_ref[...],
        preferred_element_type=jnp.float32,
        precision=jax.lax.Precision.DEFAULT,
    ) + b_ref[...]
    if relu:
        acc = jnp.maximum(acc, 0.0)
    o_ref[...] = acc


def _aggregate(adj, s, b, relu, bi):
    n = adj.shape[0]
    h = s.shape[1]
    return pl.pallas_call(
        functools.partial(_agg_body, relu=relu),
        grid=(n // bi,),
        in_specs=[
            pl.BlockSpec((bi, n), lambda i: (i, 0)),
            pl.BlockSpec((n, h), lambda i: (0, 0)),
            pl.BlockSpec((1, h), lambda i: (0, 0)),
        ],
        out_specs=pl.BlockSpec((bi, h), lambda i: (i, 0)),
        out_shape=jax.ShapeDtypeStruct((n, h), jnp.float32),
        compiler_params=pltpu.CompilerParams(
            dimension_semantics=("arbitrary",)
        ),
    )(adj, s, b)


def kernel(x, adj, W1, b1, W2, b2):
    s1 = _dense(x, W1)
    x1 = _aggregate(adj, s1, b1.reshape(1, -1), True, 400)
    s2 = _dense(x1, W2)
    x2 = _aggregate(adj, s2, b2.reshape(1, -1), False, 400)
    return (x1, x2)
